# Initial kernel scaffold; baseline (speedup 1.0000x reference)
import jax
import jax.numpy as jnp
from jax.experimental import pallas as pl


def kernel(embeddings, softmax_w, softmax_b, targets):
    raise NotImplementedError("write your pallas kernel here")


# speedup vs baseline: 1.2696x; 1.2696x over previous
import functools

import jax
import jax.numpy as jnp
from jax.experimental import pallas as pl
from jax.experimental.pallas import tpu as pltpu


_NEG_HUGE = -1.0e30


def _round_up(x, m):
    return (x + m - 1) // m * m


def _loss_kernel(emb_ref, w_ref, b_ref, tgt_ref, out_ref,
                 emb_sc, m_sc, l_sc, t_sc, *, tv):
    j = pl.program_id(1)

    @pl.when(j == 0)
    def _():
        emb_sc[...] = emb_ref[...].astype(jnp.bfloat16)
        m_sc[...] = jnp.full_like(m_sc, -jnp.inf)
        l_sc[...] = jnp.zeros_like(l_sc)
        t_sc[...] = jnp.zeros_like(t_sc)

    logits = jnp.dot(emb_sc[...], w_ref[...].astype(jnp.bfloat16),
                     preferred_element_type=jnp.float32) + b_ref[...]

    m_prev = m_sc[...]
    m_new = jnp.maximum(m_prev, logits.max(axis=-1, keepdims=True))
    l_sc[...] = (l_sc[...] * jnp.exp(m_prev - m_new)
                 + jnp.sum(jnp.exp(logits - m_new), axis=-1, keepdims=True))
    m_sc[...] = m_new

    bn = logits.shape[0]
    col = jax.lax.broadcasted_iota(jnp.int32, (bn, tv), 1)
    match = col == (tgt_ref[0, 0, :][:, None] - j * tv)
    t_sc[...] += jnp.sum(jnp.where(match, logits, 0.0), axis=-1, keepdims=True)

    @pl.when(j == pl.num_programs(1) - 1)
    def _():
        out_ref[...] = m_sc[...] + jnp.log(l_sc[...]) - t_sc[...]


def kernel(embeddings, softmax_w, softmax_b, targets):
    n, d = embeddings.shape
    d2, v = softmax_w.shape
    assert d == d2

    tv = 1280
    if v % tv != 0:
        for cand in (1024, 768, 512, 384, 256, 128):
            if v % cand == 0:
                tv = cand
                break
    v_pad = _round_up(v, tv)

    n_pad = _round_up(max(n, 8), 256)
    bn = n_pad // 2 if n_pad % 512 == 0 else n_pad
    nb = n_pad // bn

    emb_p = embeddings
    if n_pad != n:
        emb_p = jnp.zeros((n_pad, d), embeddings.dtype).at[:n].set(embeddings)
    if v_pad != v:
        w_p = jnp.zeros((d, v_pad), softmax_w.dtype).at[:, :v].set(softmax_w)
        b_p = jnp.full((v_pad,), _NEG_HUGE, jnp.float32).at[:v].set(
            softmax_b.astype(jnp.float32))
    else:
        w_p = softmax_w
        b_p = softmax_b.astype(jnp.float32)
    b2d = b_p.reshape(1, v_pad)

    tgt = jnp.zeros((n_pad,), jnp.int32).at[:n].set(targets.astype(jnp.int32))
    tgt3 = tgt.reshape(nb, 1, bn)

    per_row = pl.pallas_call(
        functools.partial(_loss_kernel, tv=tv),
        out_shape=jax.ShapeDtypeStruct((n_pad, 1), jnp.float32),
        grid_spec=pltpu.PrefetchScalarGridSpec(
            num_scalar_prefetch=0,
            grid=(nb, v_pad // tv),
            in_specs=[
                pl.BlockSpec((bn, d), lambda i, j: (i, 0)),
                pl.BlockSpec((d, tv), lambda i, j: (0, j)),
                pl.BlockSpec((1, tv), lambda i, j: (0, j)),
                pl.BlockSpec((1, 1, bn), lambda i, j: (i, 0, 0)),
            ],
            out_specs=pl.BlockSpec((bn, 1), lambda i, j: (i, 0)),
            scratch_shapes=[
                pltpu.VMEM((bn, d), jnp.bfloat16),
                pltpu.VMEM((bn, 1), jnp.float32),
                pltpu.VMEM((bn, 1), jnp.float32),
                pltpu.VMEM((bn, 1), jnp.float32),
            ],
        ),
        compiler_params=pltpu.CompilerParams(
            dimension_semantics=("parallel", "arbitrary"),
            vmem_limit_bytes=64 * 1024 * 1024),
    )(emb_p, w_p, b2d, tgt3)

    return jnp.sum(per_row[:n, 0])
